# baseline (speedup 1.0000x reference)
import jax
import jax.numpy as jnp
from jax.experimental import pallas as pl


def _agg_body(self_ref, emb_ref, mask_ref, neigh_ref, e_ref, w13_ref,
              w2t_ref, b_ref, out_ref):
    nv = neigh_ref[...]
    m = mask_ref[...]
    d = self_ref.shape[1]
    m_exp = jnp.dot(m, e_ref[...], preferred_element_type=jnp.float32)
    p = nv * m_exp
    w13 = w13_ref[...]
    acc = jnp.dot(p, w2t_ref[...], preferred_element_type=jnp.float32)
    acc = acc + jnp.dot(self_ref[...], w13[0:d],
                        preferred_element_type=jnp.float32)
    acc = acc + jnp.dot(emb_ref[...], w13[d:2 * d],
                        preferred_element_type=jnp.float32)
    out_ref[...] = jnp.maximum(acc + b_ref[...], 0.0)


def kernel(self_vectors, neighbor_vectors, masks, node_emb, W, b):
    B_, _, H_, D_ = self_vectors.shape
    N_ = neighbor_vectors.shape[2]
    O_ = W.shape[1]
    BH = B_ * H_
    sv = self_vectors.reshape(BH, D_)
    nv2 = neighbor_vectors.reshape(BH, N_ * D_)
    mk = masks.reshape(BH, N_)
    ne = node_emb.reshape(BH, D_)
    b2 = b.reshape(1, O_)
    w13 = jnp.concatenate([W[:D_], W[2 * D_:]], axis=0)
    w2t = jnp.tile(W[D_:2 * D_] * (1.0 / N_), (N_, 1))
    e_mat = jnp.repeat(jnp.eye(N_, dtype=jnp.float32), D_, axis=1)

    R = 1024
    grid = (BH // R,)
    out = pl.pallas_call(
        _agg_body,
        grid=grid,
        in_specs=[
            pl.BlockSpec((R, D_), lambda i: (i, 0)),
            pl.BlockSpec((R, D_), lambda i: (i, 0)),
            pl.BlockSpec((R, N_), lambda i: (i, 0)),
            pl.BlockSpec((R, N_ * D_), lambda i: (i, 0)),
            pl.BlockSpec((N_, N_ * D_), lambda i: (0, 0)),
            pl.BlockSpec((2 * D_, O_), lambda i: (0, 0)),
            pl.BlockSpec((N_ * D_, O_), lambda i: (0, 0)),
            pl.BlockSpec((1, O_), lambda i: (0, 0)),
        ],
        out_specs=pl.BlockSpec((R, O_), lambda i: (i, 0)),
        out_shape=jax.ShapeDtypeStruct((BH, O_), jnp.float32),
    )(sv, ne, mk, nv2, e_mat, w13, w2t, b2)
    return out.reshape(B_, 1, H_, O_)


# speedup vs baseline: 2.5866x; 2.5866x over previous
import jax
import jax.numpy as jnp
from jax.experimental import pallas as pl


def _agg_body(self_ref, emb_ref, mask_ref, neigh_ref, w_ref, b_ref, out_ref):
    nv = neigh_ref[...]
    m = mask_ref[...]
    mean = jnp.sum(nv * m[:, :, None], axis=1)
    w = w_ref[...]
    d = mean.shape[1]
    acc = jnp.dot(self_ref[...], w[0:d], preferred_element_type=jnp.float32)
    acc = acc + jnp.dot(mean, w[d:2 * d], preferred_element_type=jnp.float32)
    acc = acc + jnp.dot(emb_ref[...], w[2 * d:3 * d],
                        preferred_element_type=jnp.float32)
    out_ref[...] = jnp.maximum(acc + b_ref[...], 0.0)


def kernel(self_vectors, neighbor_vectors, masks, node_emb, W, b):
    B_, _, H_, D_ = self_vectors.shape
    N_ = neighbor_vectors.shape[2]
    O_ = W.shape[1]
    BH = B_ * H_
    sv = self_vectors.reshape(BH, D_)
    nv = neighbor_vectors.reshape(BH, N_, D_)
    mk = masks.reshape(BH, N_)
    ne = node_emb.reshape(BH, D_)
    b2 = b.reshape(1, O_)
    w_scaled = jnp.concatenate(
        [W[:D_], W[D_:2 * D_] * (1.0 / N_), W[2 * D_:]], axis=0)

    R = 1024
    grid = (BH // R,)
    out = pl.pallas_call(
        _agg_body,
        grid=grid,
        in_specs=[
            pl.BlockSpec((R, D_), lambda i: (i, 0)),
            pl.BlockSpec((R, D_), lambda i: (i, 0)),
            pl.BlockSpec((R, N_), lambda i: (i, 0)),
            pl.BlockSpec((R, N_, D_), lambda i: (i, 0, 0)),
            pl.BlockSpec((3 * D_, O_), lambda i: (0, 0)),
            pl.BlockSpec((1, O_), lambda i: (0, 0)),
        ],
        out_specs=pl.BlockSpec((R, O_), lambda i: (i, 0)),
        out_shape=jax.ShapeDtypeStruct((BH, O_), jnp.float32),
    )(sv, ne, mk, nv, w_scaled, b2)
    return out.reshape(B_, 1, H_, O_)


# speedup vs baseline: 2.6508x; 1.0248x over previous
import jax
import jax.numpy as jnp
from jax.experimental import pallas as pl


def _agg_body(self_ref, emb_ref, mask_ref, neigh_ref, w_ref, b_ref, out_ref):
    nv = neigh_ref[...]
    m = mask_ref[...] * (1.0 / nv.shape[1])
    mean = jnp.sum(nv * m[:, :, None], axis=1)
    w = w_ref[...]
    d = mean.shape[1]
    acc = jnp.dot(self_ref[...], w[0:d], preferred_element_type=jnp.float32)
    acc = acc + jnp.dot(mean, w[d:2 * d], preferred_element_type=jnp.float32)
    acc = acc + jnp.dot(emb_ref[...], w[2 * d:3 * d],
                        preferred_element_type=jnp.float32)
    out_ref[...] = jnp.maximum(acc + b_ref[...], 0.0)


def kernel(self_vectors, neighbor_vectors, masks, node_emb, W, b):
    B_, _, H_, D_ = self_vectors.shape
    N_ = neighbor_vectors.shape[2]
    O_ = W.shape[1]
    BH = B_ * H_
    sv = self_vectors.reshape(BH, D_)
    nv = neighbor_vectors.reshape(BH, N_, D_)
    mk = masks.reshape(BH, N_)
    ne = node_emb.reshape(BH, D_)
    b2 = b.reshape(1, O_)

    R = 1024
    grid = (BH // R,)
    out = pl.pallas_call(
        _agg_body,
        grid=grid,
        in_specs=[
            pl.BlockSpec((R, D_), lambda i: (i, 0)),
            pl.BlockSpec((R, D_), lambda i: (i, 0)),
            pl.BlockSpec((R, N_), lambda i: (i, 0)),
            pl.BlockSpec((R, N_, D_), lambda i: (i, 0, 0)),
            pl.BlockSpec((3 * D_, O_), lambda i: (0, 0)),
            pl.BlockSpec((1, O_), lambda i: (0, 0)),
        ],
        out_specs=pl.BlockSpec((R, O_), lambda i: (i, 0)),
        out_shape=jax.ShapeDtypeStruct((BH, O_), jnp.float32),
    )(sv, ne, mk, nv, W, b2)
    return out.reshape(B_, 1, H_, O_)


# speedup vs baseline: 2.7465x; 1.0361x over previous
import jax
import jax.numpy as jnp
from jax.experimental import pallas as pl


def _agg_body(self_ref, emb_ref, mask_ref, neigh_ref, w_ref, b_ref, out_ref):
    nv = neigh_ref[...]
    m = mask_ref[...] * (1.0 / nv.shape[1])
    mean = jnp.sum(nv * m[:, :, None], axis=1)
    w = w_ref[...]
    d = mean.shape[1]
    acc = jnp.dot(self_ref[...], w[0:d], preferred_element_type=jnp.float32)
    acc = acc + jnp.dot(mean, w[d:2 * d], preferred_element_type=jnp.float32)
    acc = acc + jnp.dot(emb_ref[...], w[2 * d:3 * d],
                        preferred_element_type=jnp.float32)
    out_ref[...] = jnp.maximum(acc + b_ref[...], 0.0)


def kernel(self_vectors, neighbor_vectors, masks, node_emb, W, b):
    B_, _, H_, D_ = self_vectors.shape
    N_ = neighbor_vectors.shape[2]
    O_ = W.shape[1]
    BH = B_ * H_
    sv = self_vectors.reshape(BH, D_)
    nv = neighbor_vectors.reshape(BH, N_, D_)
    mk = masks.reshape(BH, N_)
    ne = node_emb.reshape(BH, D_)
    b2 = b.reshape(1, O_)

    R = 2048
    grid = (BH // R,)
    out = pl.pallas_call(
        _agg_body,
        grid=grid,
        in_specs=[
            pl.BlockSpec((R, D_), lambda i: (i, 0)),
            pl.BlockSpec((R, D_), lambda i: (i, 0)),
            pl.BlockSpec((R, N_), lambda i: (i, 0)),
            pl.BlockSpec((R, N_, D_), lambda i: (i, 0, 0)),
            pl.BlockSpec((3 * D_, O_), lambda i: (0, 0)),
            pl.BlockSpec((1, O_), lambda i: (0, 0)),
        ],
        out_specs=pl.BlockSpec((R, O_), lambda i: (i, 0)),
        out_shape=jax.ShapeDtypeStruct((BH, O_), jnp.float32),
    )(sv, ne, mk, nv, W, b2)
    return out.reshape(B_, 1, H_, O_)
